# baseline (speedup 1.0000x reference)
import jax
import jax.numpy as jnp
from jax.experimental import pallas as pl
from jax.experimental.pallas import tpu as pltpu
from jax.experimental.pallas import tpu_sc as plsc

E = 8
TOPK = 2
D = 2048
F = 1024
T = 2048
ALPHA = 7.0
LANES = 128
SEG = D // 128

BLK = 128
NB = 39
NP = NB * BLK
A = TOPK * T


def _router_body(x_ref, rwt_ref, bias_ref, xc_ref, pos1_ref, pos2_ref,
                 w1_ref, w2_ref, ex_ref, nb_ref):
    x = x_ref[...]
    xc_ref[...] = x.reshape(T * SEG, 128)

    logits = jnp.dot(x, rwt_ref[...], preferred_element_type=jnp.float32)
    scores = jnp.sqrt(jnp.logaddexp(logits, 0.0))
    li = jax.lax.broadcasted_iota(jnp.int32, (T, LANES), 1)
    valid = li < E
    neg = jnp.float32(-jnp.inf)
    sel = jnp.where(valid, scores + bias_ref[...], neg)
    m1 = jnp.max(sel, axis=1, keepdims=True)
    i1 = jnp.min(jnp.where(sel == m1, li, LANES), axis=1, keepdims=True)
    sel2 = jnp.where(li == i1, neg, sel)
    m2 = jnp.max(sel2, axis=1, keepdims=True)
    i2 = jnp.min(jnp.where(sel2 == m2, li, LANES), axis=1, keepdims=True)
    s1 = jnp.sum(jnp.where(li == i1, scores, 0.0), axis=1, keepdims=True)
    s2 = jnp.sum(jnp.where(li == i2, scores, 0.0), axis=1, keepdims=True)
    denom = s1 + s2 + 1e-20
    w1_ref[...] = s1 / denom
    w2_ref[...] = s2 / denom

    mask = ((li == i1) | (li == i2)).astype(jnp.bfloat16)
    ri_t = jax.lax.broadcasted_iota(jnp.int32, (T, T), 0)
    ci_t = jax.lax.broadcasted_iota(jnp.int32, (T, T), 1)
    tril = (ci_t <= ri_t).astype(jnp.bfloat16)
    rank_inc = jnp.dot(tril, mask, preferred_element_type=jnp.float32)
    rank_ex = rank_inc - mask.astype(jnp.float32)

    counts = rank_inc[T - 1:T, :]
    pblk = jnp.ceil(counts * (1.0 / BLK)) * BLK

    ru = jax.lax.broadcasted_iota(jnp.int32, (LANES, LANES), 0)
    cu = jax.lax.broadcasted_iota(jnp.int32, (LANES, LANES), 1)
    upper = ((ru < cu) & (ru < E)).astype(jnp.float32)
    offs = jnp.round(jax.lax.dot_general(
        pblk, upper, (((1,), (0,)), ((), ())),
        precision=jax.lax.Precision.HIGHEST,
        preferred_element_type=jnp.float32))

    slot = offs + rank_ex
    pos1 = jnp.sum(jnp.where(li == i1, slot, 0.0), axis=1, keepdims=True)
    pos2 = jnp.sum(jnp.where(li == i2, slot, 0.0), axis=1, keepdims=True)
    pos1_ref[...] = pos1.astype(jnp.int32)
    pos2_ref[...] = pos2.astype(jnp.int32)

    ends = offs + pblk
    rb = jax.lax.broadcasted_iota(jnp.int32, (LANES, LANES), 0)
    cb = jax.lax.broadcasted_iota(jnp.int32, (LANES, LANES), 1)
    ends_b = jnp.broadcast_to(ends, (LANES, LANES))
    ind = ((rb * BLK >= ends_b) & (cb < E)).astype(jnp.float32)
    ex = jnp.sum(ind, axis=1, keepdims=True)
    ex_ref[...] = jnp.minimum(ex, E - 1).astype(jnp.int32)
    nblocks = jnp.sum(pblk * (1.0 / BLK), axis=1, keepdims=True)
    nb_ref[...] = jnp.broadcast_to(nblocks, (LANES, 1)).astype(jnp.int32)


def _route(flat, rw_pad, bias_pad):
    return pl.pallas_call(
        _router_body,
        out_shape=(
            jax.ShapeDtypeStruct((T * SEG, 128), jnp.float32),
            jax.ShapeDtypeStruct((T, 1), jnp.int32),
            jax.ShapeDtypeStruct((T, 1), jnp.int32),
            jax.ShapeDtypeStruct((T, 1), jnp.float32),
            jax.ShapeDtypeStruct((T, 1), jnp.float32),
            jax.ShapeDtypeStruct((LANES, 1), jnp.int32),
            jax.ShapeDtypeStruct((LANES, 1), jnp.int32),
        ),
    )(flat, rw_pad, bias_pad)


SC_WIN = 128
_SC_MESH = dict(core_axis_name="core", subcore_axis_name="subcore")


def _sc_scatter(x16, idx16):
    nseg = T * SEG
    ntb = nseg // SC_WIN

    @pl.kernel(out_type=jax.ShapeDtypeStruct((NP * SEG, 128), jnp.float32),
               mesh=plsc.VectorSubcoreMesh(**_SC_MESH))
    def _kernel(x_hbm, i_hbm, o_hbm):
        def body(x_vmem, i_vmem):
            pltpu.sync_copy(x_vmem, o_hbm.at[i_vmem.at[0]])

        pltpu.emit_pipeline(
            body,
            grid=(TOPK * nseg // SC_WIN,),
            in_specs=[
                pl.BlockSpec((SC_WIN, 128), index_map=lambda i: (i % ntb, 0)),
                pl.BlockSpec((1, SC_WIN), index_map=lambda i: (0, i)),
            ],
            out_specs=[],
            core_axis_name=("core", "subcore"),
            dimension_semantics=(pltpu.PARALLEL,),
        )(x_hbm, i_hbm)

    return _kernel(x16, idx16)


def _sc_gather(ys16, idx16):
    @pl.kernel(out_type=jax.ShapeDtypeStruct((A * SEG, 128), jnp.float32),
               mesh=plsc.VectorSubcoreMesh(**_SC_MESH))
    def _kernel(y_hbm, i_hbm, o_hbm):
        def body(i_vmem, o_vmem):
            pltpu.sync_copy(y_hbm.at[i_vmem.at[0]], o_vmem)

        pltpu.emit_pipeline(
            body,
            grid=(A * SEG // SC_WIN,),
            in_specs=[pl.BlockSpec((1, SC_WIN), index_map=lambda i: (0, i))],
            out_specs=[pl.BlockSpec((SC_WIN, 128), index_map=lambda i: (i, 0))],
            core_axis_name=("core", "subcore"),
            dimension_semantics=(pltpu.PARALLEL,),
        )(i_hbm, o_hbm)

    return _kernel(ys16, idx16)


SBT = 1024
SBF = 128


def _shared_body(x_ref, wg_ref, wu_ref, wd_ref, o_ref, acc):
    f = pl.program_id(1)
    x = x_ref[...].astype(jnp.bfloat16)
    g = jnp.dot(x, wg_ref[...].astype(jnp.bfloat16),
                preferred_element_type=jnp.float32)
    u = jnp.dot(x, wu_ref[...].astype(jnp.bfloat16),
                preferred_element_type=jnp.float32)
    g = jnp.minimum(g, ALPHA)
    u = jnp.clip(u, -ALPHA, ALPHA)
    act = (g * jax.nn.sigmoid(g) * u).astype(jnp.bfloat16)
    y = jnp.dot(act, wd_ref[...].astype(jnp.bfloat16),
                preferred_element_type=jnp.float32)

    nf = F // SBF

    @pl.when(f == 0)
    def _():
        acc[...] = y

    @pl.when((f > 0) & (f < nf - 1))
    def _():
        acc[...] += y

    @pl.when(f == nf - 1)
    def _():
        o_ref[...] = acc[...] + y


def _shared_ffn(xb, SWg, SWu, SWd):
    return pl.pallas_call(
        _shared_body,
        grid=(T // SBT, F // SBF),
        in_specs=[
            pl.BlockSpec((SBT, D), lambda t, f: (t, 0)),
            pl.BlockSpec((D, SBF), lambda t, f: (0, f)),
            pl.BlockSpec((D, SBF), lambda t, f: (0, f)),
            pl.BlockSpec((SBF, D), lambda t, f: (f, 0)),
        ],
        out_specs=pl.BlockSpec((SBT, D), lambda t, f: (t, 0)),
        out_shape=jax.ShapeDtypeStruct((T, D), jnp.float32),
        scratch_shapes=[pltpu.VMEM((SBT, D), jnp.float32)],
    )(xb, SWg, SWu, SWd)


GBF = 512


def _gffn_body(s_ref, x_ref, wg_ref, wu_ref, wd_ref, y_ref):
    b = pl.program_id(0)

    @pl.when(b < s_ref[NB])
    def _():
        x = x_ref[...].reshape(BLK, D).astype(jnp.bfloat16)
        g = jnp.dot(x, wg_ref[0].astype(jnp.bfloat16),
                    preferred_element_type=jnp.float32)
        u = jnp.dot(x, wu_ref[0].astype(jnp.bfloat16),
                    preferred_element_type=jnp.float32)
        g = jnp.minimum(g, ALPHA)
        u = jnp.clip(u, -ALPHA, ALPHA)
        act = (g * jax.nn.sigmoid(g) * u).astype(jnp.bfloat16)
        y = jnp.dot(act, wd_ref[0].astype(jnp.bfloat16),
                    preferred_element_type=jnp.float32)
        y_ref[...] = y.reshape(BLK * SEG, 128)


def _grouped_ffn(scalars, xs, Wg, Wu, Wd):
    grid_spec = pltpu.PrefetchScalarGridSpec(
        num_scalar_prefetch=1,
        grid=(NB,),
        in_specs=[
            pl.BlockSpec((BLK * SEG, 128), lambda b, s: (b, 0)),
            pl.BlockSpec((1, D, F), lambda b, s: (s[b], 0, 0)),
            pl.BlockSpec((1, D, F), lambda b, s: (s[b], 0, 0)),
            pl.BlockSpec((1, F, D), lambda b, s: (s[b], 0, 0)),
        ],
        out_specs=pl.BlockSpec((BLK * SEG, 128), lambda b, s: (b, 0)),
    )
    return pl.pallas_call(
        _gffn_body,
        grid_spec=grid_spec,
        out_shape=jax.ShapeDtypeStruct((NP * SEG, 128), jnp.float32),
    )(scalars, xs, Wg, Wu, Wd)


CBT = 512


def _combine_body(g1_ref, g2_ref, ysh_ref, w1_ref, w2_ref, o_ref):
    g1 = g1_ref[...].reshape(CBT, D)
    g2 = g2_ref[...].reshape(CBT, D)
    o_ref[...] = w1_ref[...] * g1 + w2_ref[...] * g2 + ysh_ref[...]


def _combine(g, ysh, w1, w2):
    nt = T // CBT
    return pl.pallas_call(
        _combine_body,
        grid=(nt,),
        in_specs=[
            pl.BlockSpec((CBT * SEG, 128), lambda t: (t, 0)),
            pl.BlockSpec((CBT * SEG, 128), lambda t: (t + nt, 0)),
            pl.BlockSpec((CBT, D), lambda t: (t, 0)),
            pl.BlockSpec((CBT, 1), lambda t: (t, 0)),
            pl.BlockSpec((CBT, 1), lambda t: (t, 0)),
        ],
        out_specs=pl.BlockSpec((CBT, D), lambda t: (t, 0)),
        out_shape=jax.ShapeDtypeStruct((T, D), jnp.float32),
    )(g, g, ysh, w1, w2)


@jax.jit
def kernel(hidden, router_w, expert_bias, Wg, Wu, Wd, SWg, SWu, SWd):
    B, S, Dm = hidden.shape
    flat = hidden.reshape(T, D)

    rw_pad = jnp.zeros((D, LANES), jnp.float32).at[:, :E].set(router_w.T)
    bias_pad = jnp.zeros((1, LANES), jnp.float32).at[:, :E].set(
        expert_bias[None, :])

    xc, pos1, pos2, w1, w2, ex_col, nb_col = _route(flat, rw_pad, bias_pad)

    scalars = jnp.concatenate(
        [ex_col[:NB, 0], nb_col[0, 0:1]]).astype(jnp.int32)
    idx = jnp.concatenate([pos1[:, 0], pos2[:, 0]])
    idx16 = (idx[:, None] * SEG +
             jnp.arange(SEG, dtype=jnp.int32)[None, :]).reshape(1, A * SEG)

    xs = _sc_scatter(xc, idx16)
    ys = _grouped_ffn(scalars, xs, Wg, Wu, Wd)
    flat_b, ys_b = jax.lax.optimization_barrier((flat, ys))
    ysh = _shared_ffn(flat_b, SWg, SWu, SWd)
    g = _sc_gather(ys_b, idx16)
    out = _combine(g, ysh, w1, w2)
    return out.reshape(B, S, Dm)


# speedup vs baseline: 1.0212x; 1.0212x over previous
import jax
import jax.numpy as jnp
from jax.experimental import pallas as pl
from jax.experimental.pallas import tpu as pltpu
from jax.experimental.pallas import tpu_sc as plsc

E = 8
TOPK = 2
D = 2048
F = 1024
T = 2048
ALPHA = 7.0
LANES = 128
SEG = D // 128

BLK = 128
NB = 39
NP = NB * BLK
A = TOPK * T


def _router_body(x_ref, rwt_ref, bias_ref, xc_ref, pos1_ref, pos2_ref,
                 w1_ref, w2_ref, ex_ref, nb_ref):
    x = x_ref[...]
    xc_ref[...] = x.reshape(T * SEG, 128)

    logits = jnp.dot(x, rwt_ref[...], preferred_element_type=jnp.float32)
    scores = jnp.sqrt(jnp.logaddexp(logits, 0.0))
    li = jax.lax.broadcasted_iota(jnp.int32, (T, LANES), 1)
    valid = li < E
    neg = jnp.float32(-jnp.inf)
    sel = jnp.where(valid, scores + bias_ref[...], neg)
    m1 = jnp.max(sel, axis=1, keepdims=True)
    i1 = jnp.min(jnp.where(sel == m1, li, LANES), axis=1, keepdims=True)
    sel2 = jnp.where(li == i1, neg, sel)
    m2 = jnp.max(sel2, axis=1, keepdims=True)
    i2 = jnp.min(jnp.where(sel2 == m2, li, LANES), axis=1, keepdims=True)
    s1 = jnp.sum(jnp.where(li == i1, scores, 0.0), axis=1, keepdims=True)
    s2 = jnp.sum(jnp.where(li == i2, scores, 0.0), axis=1, keepdims=True)
    denom = s1 + s2 + 1e-20
    w1_ref[...] = s1 / denom
    w2_ref[...] = s2 / denom

    mask = ((li == i1) | (li == i2)).astype(jnp.bfloat16)
    ri_t = jax.lax.broadcasted_iota(jnp.int32, (T, T), 0)
    ci_t = jax.lax.broadcasted_iota(jnp.int32, (T, T), 1)
    tril = (ci_t <= ri_t).astype(jnp.bfloat16)
    rank_inc = jnp.dot(tril, mask, preferred_element_type=jnp.float32)
    rank_ex = rank_inc - mask.astype(jnp.float32)

    counts = rank_inc[T - 1:T, :]
    pblk = jnp.ceil(counts * (1.0 / BLK)) * BLK

    ru = jax.lax.broadcasted_iota(jnp.int32, (LANES, LANES), 0)
    cu = jax.lax.broadcasted_iota(jnp.int32, (LANES, LANES), 1)
    upper = ((ru < cu) & (ru < E)).astype(jnp.float32)
    offs = jnp.round(jax.lax.dot_general(
        pblk, upper, (((1,), (0,)), ((), ())),
        precision=jax.lax.Precision.HIGHEST,
        preferred_element_type=jnp.float32))

    slot = offs + rank_ex
    pos1 = jnp.sum(jnp.where(li == i1, slot, 0.0), axis=1, keepdims=True)
    pos2 = jnp.sum(jnp.where(li == i2, slot, 0.0), axis=1, keepdims=True)
    pos1_ref[...] = pos1.astype(jnp.int32)
    pos2_ref[...] = pos2.astype(jnp.int32)

    ends = offs + pblk
    rb = jax.lax.broadcasted_iota(jnp.int32, (LANES, LANES), 0)
    cb = jax.lax.broadcasted_iota(jnp.int32, (LANES, LANES), 1)
    ends_b = jnp.broadcast_to(ends, (LANES, LANES))
    ind = ((rb * BLK >= ends_b) & (cb < E)).astype(jnp.float32)
    ex = jnp.sum(ind, axis=1, keepdims=True)
    ex_ref[...] = jnp.minimum(ex, E - 1).astype(jnp.int32)
    nblocks = jnp.sum(pblk * (1.0 / BLK), axis=1, keepdims=True)
    nb_ref[...] = jnp.broadcast_to(nblocks, (LANES, 1)).astype(jnp.int32)


def _route(flat, rw_pad, bias_pad):
    return pl.pallas_call(
        _router_body,
        out_shape=(
            jax.ShapeDtypeStruct((T * SEG, 128), jnp.float32),
            jax.ShapeDtypeStruct((T, 1), jnp.int32),
            jax.ShapeDtypeStruct((T, 1), jnp.int32),
            jax.ShapeDtypeStruct((T, 1), jnp.float32),
            jax.ShapeDtypeStruct((T, 1), jnp.float32),
            jax.ShapeDtypeStruct((LANES, 1), jnp.int32),
            jax.ShapeDtypeStruct((LANES, 1), jnp.int32),
        ),
    )(flat, rw_pad, bias_pad)


SC_WIN = 128
_SC_MESH = dict(core_axis_name="core", subcore_axis_name="subcore")


def _sc_scatter(x16, idx16):
    nseg = T * SEG
    ntb = nseg // SC_WIN

    @pl.kernel(out_type=jax.ShapeDtypeStruct((NP * SEG, 128), jnp.float32),
               mesh=plsc.VectorSubcoreMesh(**_SC_MESH))
    def _kernel(x_hbm, i_hbm, o_hbm):
        def body(x_vmem, i_vmem):
            pltpu.sync_copy(x_vmem, o_hbm.at[i_vmem.at[0]])

        pltpu.emit_pipeline(
            body,
            grid=(TOPK * nseg // SC_WIN,),
            in_specs=[
                pl.BlockSpec((SC_WIN, 128), index_map=lambda i: (i % ntb, 0)),
                pl.BlockSpec((1, SC_WIN), index_map=lambda i: (0, i)),
            ],
            out_specs=[],
            core_axis_name=("core", "subcore"),
            dimension_semantics=(pltpu.PARALLEL,),
        )(x_hbm, i_hbm)

    return _kernel(x16, idx16)


def _sc_gather(ys16, idx16):
    @pl.kernel(out_type=jax.ShapeDtypeStruct((A * SEG // 2, 128), jnp.int32),
               mesh=plsc.VectorSubcoreMesh(**_SC_MESH))
    def _kernel(y_hbm, i_hbm, o_hbm):
        def body(i_vmem, o_vmem):
            pltpu.sync_copy(y_hbm.at[i_vmem.at[0]], o_vmem)

        pltpu.emit_pipeline(
            body,
            grid=(A * SEG // 2 // SC_WIN,),
            in_specs=[pl.BlockSpec((1, SC_WIN), index_map=lambda i: (0, i))],
            out_specs=[pl.BlockSpec((SC_WIN, 128), index_map=lambda i: (i, 0))],
            core_axis_name=("core", "subcore"),
            dimension_semantics=(pltpu.PARALLEL,),
        )(i_hbm, o_hbm)

    return _kernel(ys16, idx16)


SBT = 1024
SBF = 128


def _shared_body(x_ref, wg_ref, wu_ref, wd_ref, o_ref, acc):
    f = pl.program_id(1)
    x = x_ref[...].astype(jnp.bfloat16)
    g = jnp.dot(x, wg_ref[...].astype(jnp.bfloat16),
                preferred_element_type=jnp.float32)
    u = jnp.dot(x, wu_ref[...].astype(jnp.bfloat16),
                preferred_element_type=jnp.float32)
    g = jnp.minimum(g, ALPHA)
    u = jnp.clip(u, -ALPHA, ALPHA)
    act = (g * jax.nn.sigmoid(g) * u).astype(jnp.bfloat16)
    y = jnp.dot(act, wd_ref[...].astype(jnp.bfloat16),
                preferred_element_type=jnp.float32)

    nf = F // SBF

    @pl.when(f == 0)
    def _():
        acc[...] = y

    @pl.when((f > 0) & (f < nf - 1))
    def _():
        acc[...] += y

    @pl.when(f == nf - 1)
    def _():
        o_ref[...] = acc[...] + y


def _shared_ffn(xb, SWg, SWu, SWd):
    return pl.pallas_call(
        _shared_body,
        grid=(T // SBT, F // SBF),
        in_specs=[
            pl.BlockSpec((SBT, D), lambda t, f: (t, 0)),
            pl.BlockSpec((D, SBF), lambda t, f: (0, f)),
            pl.BlockSpec((D, SBF), lambda t, f: (0, f)),
            pl.BlockSpec((SBF, D), lambda t, f: (f, 0)),
        ],
        out_specs=pl.BlockSpec((SBT, D), lambda t, f: (t, 0)),
        out_shape=jax.ShapeDtypeStruct((T, D), jnp.float32),
        scratch_shapes=[pltpu.VMEM((SBT, D), jnp.float32)],
    )(xb, SWg, SWu, SWd)


GBF = 512


def _gffn_body(s_ref, x_ref, wg_ref, wu_ref, wd_ref, y_ref):
    b = pl.program_id(0)

    @pl.when(b < s_ref[NB])
    def _():
        x = x_ref[...].reshape(BLK, D).astype(jnp.bfloat16)
        g = jnp.dot(x, wg_ref[0].astype(jnp.bfloat16),
                    preferred_element_type=jnp.float32)
        u = jnp.dot(x, wu_ref[0].astype(jnp.bfloat16),
                    preferred_element_type=jnp.float32)
        g = jnp.minimum(g, ALPHA)
        u = jnp.clip(u, -ALPHA, ALPHA)
        act = (g * jax.nn.sigmoid(g) * u).astype(jnp.bfloat16)
        y = jnp.dot(act, wd_ref[0].astype(jnp.bfloat16),
                    preferred_element_type=jnp.float32)
        u1 = jax.lax.bitcast_convert_type(y[:, :D // 2], jnp.int32)
        u2 = jax.lax.bitcast_convert_type(y[:, D // 2:], jnp.int32)
        r1 = ((u1 >> 16) & 1) + 0x7FFF
        r2 = ((u2 >> 16) & 1) + 0x7FFF
        hi = (u1 + r1) & jnp.int32(-65536)
        lo = ((u2 + r2) >> 16) & jnp.int32(0xFFFF)
        y_ref[...] = (hi | lo).reshape(BLK * SEG // 2, 128)


def _grouped_ffn(scalars, xs, Wg, Wu, Wd):
    grid_spec = pltpu.PrefetchScalarGridSpec(
        num_scalar_prefetch=1,
        grid=(NB,),
        in_specs=[
            pl.BlockSpec((BLK * SEG, 128), lambda b, s: (b, 0)),
            pl.BlockSpec((1, D, F), lambda b, s: (s[b], 0, 0)),
            pl.BlockSpec((1, D, F), lambda b, s: (s[b], 0, 0)),
            pl.BlockSpec((1, F, D), lambda b, s: (s[b], 0, 0)),
        ],
        out_specs=pl.BlockSpec((BLK * SEG // 2, 128), lambda b, s: (b, 0)),
    )
    return pl.pallas_call(
        _gffn_body,
        grid_spec=grid_spec,
        out_shape=jax.ShapeDtypeStruct((NP * SEG // 2, 128), jnp.int32),
    )(scalars, xs, Wg, Wu, Wd)


CBT = 512


def _combine_body(g1_ref, g2_ref, ysh_ref, w1_ref, w2_ref, o_ref):
    def _unpack(gi):
        a = jax.lax.bitcast_convert_type(
            gi & jnp.int32(-65536), jnp.float32)
        b = jax.lax.bitcast_convert_type(gi << 16, jnp.float32)
        return jnp.concatenate([a, b], axis=1)

    g1 = _unpack(g1_ref[...].reshape(CBT, D // 2))
    g2 = _unpack(g2_ref[...].reshape(CBT, D // 2))
    o_ref[...] = w1_ref[...] * g1 + w2_ref[...] * g2 + ysh_ref[...]


def _combine(g, ysh, w1, w2):
    nt = T // CBT
    return pl.pallas_call(
        _combine_body,
        grid=(nt,),
        in_specs=[
            pl.BlockSpec((CBT * SEG // 2, 128), lambda t: (t, 0)),
            pl.BlockSpec((CBT * SEG // 2, 128), lambda t: (t + nt, 0)),
            pl.BlockSpec((CBT, D), lambda t: (t, 0)),
            pl.BlockSpec((CBT, 1), lambda t: (t, 0)),
            pl.BlockSpec((CBT, 1), lambda t: (t, 0)),
        ],
        out_specs=pl.BlockSpec((CBT, D), lambda t: (t, 0)),
        out_shape=jax.ShapeDtypeStruct((T, D), jnp.float32),
    )(g, g, ysh, w1, w2)


@jax.jit
def kernel(hidden, router_w, expert_bias, Wg, Wu, Wd, SWg, SWu, SWd):
    B, S, Dm = hidden.shape
    flat = hidden.reshape(T, D)

    rw_pad = jnp.zeros((D, LANES), jnp.float32).at[:, :E].set(router_w.T)
    bias_pad = jnp.zeros((1, LANES), jnp.float32).at[:, :E].set(
        expert_bias[None, :])

    xc, pos1, pos2, w1, w2, ex_col, nb_col = _route(flat, rw_pad, bias_pad)

    scalars = jnp.concatenate(
        [ex_col[:NB, 0], nb_col[0, 0:1]]).astype(jnp.int32)
    idx = jnp.concatenate([pos1[:, 0], pos2[:, 0]])
    idx16 = (idx[:, None] * SEG +
             jnp.arange(SEG, dtype=jnp.int32)[None, :]).reshape(1, A * SEG)
    idx8 = (idx[:, None] * (SEG // 2) +
            jnp.arange(SEG // 2, dtype=jnp.int32)[None, :]
            ).reshape(1, A * SEG // 2)

    xs = _sc_scatter(xc, idx16)
    ys = _grouped_ffn(scalars, xs, Wg, Wu, Wd)
    flat_b, ys_b = jax.lax.optimization_barrier((flat, ys))
    ysh = _shared_ffn(flat_b, SWg, SWu, SWd)
    g = _sc_gather(ys_b, idx8)
    out = _combine(g, ysh, w1, w2)
    return out.reshape(B, S, Dm)
